# Initial kernel scaffold; baseline (speedup 1.0000x reference)
import jax
import jax.numpy as jnp
from jax.experimental import pallas as pl


def kernel(X_t, t_idx, edge_index, edge_attr, node_graph_idx, n_graphs, node_features, rand_nodes, params):
    raise NotImplementedError("write your pallas kernel here")


# speedup vs baseline: 1.4151x; 1.4151x over previous
import functools

import jax
import jax.numpy as jnp
import numpy as np
from jax import lax
from jax.experimental import pallas as pl
from jax.experimental.pallas import tpu as pltpu
from jax.experimental.pallas import tpu_sc as plsc

N = 50000
NP = 50176
E = 800000
EP = 819200
H = 64
EMBED_DIM = 32
DUMMY = 50050
NBLK = NP // 1024
EBLK = EP // 1024

F32 = jnp.float32


def _dot(a, b):
    return jnp.dot(a, b, preferred_element_type=F32)


def _enc_node_body(x_ref, w1_ref, b1_ref, w2_ref, b2_ref, wa_ref, ba_ref,
                   wb_ref, h_ref, a_ref, bo_ref):
    x = x_ref[...]
    z = jax.nn.relu(_dot(x, w1_ref[...]) + b1_ref[...])
    h = _dot(z, w2_ref[...]) + b2_ref[...]
    h_ref[...] = h
    a_ref[...] = _dot(h, wa_ref[...]) + ba_ref[...]
    bo_ref[...] = _dot(h, wb_ref[...])


def _enc_edge_body(x_ref, w1_ref, b1_ref, w2_ref, b2_ref, out_ref):
    z = jax.nn.relu(_dot(x_ref[...], w1_ref[...]) + b1_ref[...])
    out_ref[...] = _dot(z, w2_ref[...]) + b2_ref[...]


def _edge_body(hs_ref, hd_ref, e_ref, w1c_ref, w2_ref, b2_ref, out_ref):
    e = e_ref[...]
    z = jax.nn.relu(hs_ref[...] + hd_ref[...] + _dot(e, w1c_ref[...]))
    out_ref[...] = e + _dot(z, w2_ref[...]) + b2_ref[...]


def _node_common(h_ref, agg_ref, c2_ref, wn1h_ref, wn1a_ref, bn1_ref,
                 wn2_ref, bn2_ref):
    h = h_ref[...]
    cnt = c2_ref[0, :, 0:1] + c2_ref[1, :, 0:1]
    denom = jnp.maximum(cnt, 1.0)
    aggm = agg_ref[...] / denom
    z = jax.nn.relu(_dot(h, wn1h_ref[...]) + _dot(aggm, wn1a_ref[...])
                    + bn1_ref[...])
    return h + _dot(z, wn2_ref[...]) + bn2_ref[...]


def _node_body(h_ref, agg_ref, c2_ref, wn1h_ref, wn1a_ref, bn1_ref, wn2_ref,
               bn2_ref, wa_ref, ba_ref, wb_ref, h_out, a_out, b_out):
    hn = _node_common(h_ref, agg_ref, c2_ref, wn1h_ref, wn1a_ref, bn1_ref,
                      wn2_ref, bn2_ref)
    h_out[...] = hn
    a_out[...] = _dot(hn, wa_ref[...]) + ba_ref[...]
    b_out[...] = _dot(hn, wb_ref[...])


def _node_last_body(h_ref, agg_ref, c2_ref, wn1h_ref, wn1a_ref, bn1_ref,
                    wn2_ref, bn2_ref, wd1_ref, bd1_ref, wd2_ref, bd2_ref,
                    emb_out):
    hn = _node_common(h_ref, agg_ref, c2_ref, wn1h_ref, wn1a_ref, bn1_ref,
                      wn2_ref, bn2_ref)
    z = jax.nn.relu(_dot(hn, wd1_ref[...]) + bd1_ref[...])
    emb_out[...] = _dot(z, wd2_ref[...]) + bd2_ref[...]


def _head_body(emb_ref, g_ref, wml_ref, bml_ref, wv1_ref, bv1_ref, wv2_ref,
               bv2_ref, wv3_ref, bv3_ref, delta_ref, out8_ref, vals_ref,
               ve_scr, np_scr):
    i = pl.program_id(0)
    emb = emb_ref[...]
    o = _dot(emb, wml_ref[...]) + bml_ref[...]
    lane = lax.broadcasted_iota(jnp.int32, o.shape, 1)
    is_lv = jnp.logical_and(lane >= 2, lane < 4)
    out8_ref[...] = jnp.where(is_lv, jnp.clip(o, -10.0, 2.0), o)

    g = g_ref[0]
    gid = lax.broadcasted_iota(jnp.int32, (8, g.shape[1]), 0)
    S = (gid == g).astype(F32)

    @pl.when(i == 0)
    def _():
        ve_scr[...] = jnp.zeros_like(ve_scr)
        np_scr[...] = jnp.zeros_like(np_scr)
        vals_ref[...] = jnp.zeros_like(vals_ref)

    ve_scr[...] += _dot(S, emb)
    np_scr[...] += jnp.broadcast_to(jnp.sum(S, axis=1, keepdims=True),
                                    np_scr.shape)

    @pl.when(i == NBLK - 1)
    def _():
        n_per = np_scr[...] + delta_ref[0]
        scale = jax.lax.rsqrt(jnp.maximum(n_per, 1.0))[:, :H]
        ven = ve_scr[...] * scale
        v1 = jax.nn.relu(_dot(ven, wv1_ref[...]) + bv1_ref[...])
        v2 = jax.nn.relu(_dot(v1, wv2_ref[...]) + bv2_ref[...])
        vals_ref[...] = _dot(v2, wv3_ref[...]) + bv3_ref[...]


def _row_spec(bm, bn):
    return pl.BlockSpec((bm, bn), lambda i: (i, 0))


def _w_spec(m, n):
    return pl.BlockSpec((m, n), lambda i: (0, 0))


def _tc_enc_node(x, w1, b1, w2, b2, wa, ba, wb):
    out = [jax.ShapeDtypeStruct((NP, H), F32)] * 3
    return pl.pallas_call(
        _enc_node_body,
        grid=(NBLK,),
        in_specs=[_row_spec(1024, H), _w_spec(H, H), _w_spec(1, H),
                  _w_spec(H, H), _w_spec(1, H), _w_spec(H, H), _w_spec(1, H),
                  _w_spec(H, H)],
        out_specs=[_row_spec(1024, H)] * 3,
        out_shape=out,
    )(x, w1, b1, w2, b2, wa, ba, wb)


def _tc_enc_edge(xp, w1, b1, w2, b2):
    return pl.pallas_call(
        _enc_edge_body,
        grid=(EBLK,),
        in_specs=[_row_spec(1024, 4), _w_spec(4, H), _w_spec(1, H),
                  _w_spec(H, H), _w_spec(1, H)],
        out_specs=_row_spec(1024, H),
        out_shape=jax.ShapeDtypeStruct((EP, H), F32),
    )(xp, w1, b1, w2, b2)


def _tc_edge(hs, hd, e, w1c, w2, b2):
    return pl.pallas_call(
        _edge_body,
        grid=(EBLK,),
        in_specs=[_row_spec(1024, H)] * 3 + [_w_spec(H, H), _w_spec(H, H),
                                             _w_spec(1, H)],
        out_specs=_row_spec(1024, H),
        out_shape=jax.ShapeDtypeStruct((EP, H), F32),
    )(hs, hd, e, w1c, w2, b2)


_C2_SPEC = pl.BlockSpec((2, 1024, 16), lambda i: (0, i, 0))


def _tc_node(h, agg, c2, wn1h, wn1a, bn1, wn2, bn2, wa, ba, wb):
    out = [jax.ShapeDtypeStruct((NP, H), F32)] * 3
    return pl.pallas_call(
        _node_body,
        grid=(NBLK,),
        in_specs=[_row_spec(1024, H), _row_spec(1024, H), _C2_SPEC,
                  _w_spec(H, H), _w_spec(H, H), _w_spec(1, H), _w_spec(H, H),
                  _w_spec(1, H), _w_spec(H, H), _w_spec(1, H), _w_spec(H, H)],
        out_specs=[_row_spec(1024, H)] * 3,
        out_shape=out,
    )(h, agg, c2, wn1h, wn1a, bn1, wn2, bn2, wa, ba, wb)


def _tc_node_last(h, agg, c2, wn1h, wn1a, bn1, wn2, bn2, wd1, bd1, wd2, bd2):
    return pl.pallas_call(
        _node_last_body,
        grid=(NBLK,),
        in_specs=[_row_spec(1024, H), _row_spec(1024, H), _C2_SPEC,
                  _w_spec(H, H), _w_spec(H, H), _w_spec(1, H), _w_spec(H, H),
                  _w_spec(1, H), _w_spec(H, H), _w_spec(1, H), _w_spec(H, H),
                  _w_spec(1, H)],
        out_specs=_row_spec(1024, H),
        out_shape=jax.ShapeDtypeStruct((NP, H), F32),
    )(h, agg, c2, wn1h, wn1a, bn1, wn2, bn2, wd1, bd1, wd2, bd2)


def _tc_head(emb, g3, wml, bml, wv1, bv1, wv2, bv2, wv3, bv3, delta):
    return pl.pallas_call(
        _head_body,
        grid=(NBLK,),
        in_specs=[_row_spec(1024, H),
                  pl.BlockSpec((1, 1, 1024), lambda i: (i, 0, 0)),
                  _w_spec(H, 8), _w_spec(1, 8), _w_spec(H, 128),
                  _w_spec(1, 128), _w_spec(128, H), _w_spec(1, H),
                  _w_spec(H, 128), _w_spec(1, 128),
                  pl.BlockSpec(memory_space=pltpu.SMEM)],
        out_specs=[_row_spec(1024, 8), _w_spec(8, 128)],
        out_shape=[jax.ShapeDtypeStruct((NP, 8), F32),
                   jax.ShapeDtypeStruct((8, 128), F32)],
        scratch_shapes=[pltpu.VMEM((8, H), F32), pltpu.VMEM((8, 128), F32)],
    )(emb, g3, wml, bml, wv1, bv1, wv2, bv2, wv3, bv3, delta)


def _mesh():
    return plsc.VectorSubcoreMesh(core_axis_name="c", subcore_axis_name="s",
                                  num_cores=2, num_subcores=16)


_GK = 512
_G_PER_W = EP // 32
_G_CHUNKS = _G_PER_W // _GK


@functools.lru_cache(maxsize=None)
def _sc_gather_kernel():
    @functools.partial(
        pl.kernel,
        out_type=(jax.ShapeDtypeStruct((EP, H), F32),
                  jax.ShapeDtypeStruct((EP, H), F32)),
        mesh=_mesh(),
        compiler_params=pltpu.CompilerParams(use_tc_tiling_on_sc=False),
        scratch_types=[pltpu.VMEM((8, 128), jnp.int32),
                       pltpu.VMEM((8, 128), jnp.int32),
                       pltpu.VMEM((_GK, H), F32),
                       pltpu.VMEM((_GK, H), F32),
                       pltpu.SemaphoreType.DMA],
    )
    def k(a_hbm, b_hbm, src_hbm, dst_hbm, hs_hbm, hd_hbm,
          sidx, didx, abuf, bbuf, sem):
        wid = lax.axis_index("s") * 2 + lax.axis_index("c")

        def body(j, _):
            base = pl.multiple_of(wid * _G_PER_W + j * 1024, 1024)
            r = pl.multiple_of(base // 128, 8)
            pltpu.sync_copy(src_hbm.at[pl.ds(r, 8), :], sidx)
            pltpu.sync_copy(dst_hbm.at[pl.ds(r, 8), :], didx)
            for half in range(2):
                cps = []
                for t in range(4):
                    row = half * 4 + t
                    cps.append(pltpu.async_copy(
                        a_hbm.at[sidx.at[row]],
                        abuf.at[pl.ds(t * 128, 128)], sem))
                    cps.append(pltpu.async_copy(
                        b_hbm.at[didx.at[row]],
                        bbuf.at[pl.ds(t * 128, 128)], sem))
                for cp in cps:
                    cp.wait()
                pltpu.sync_copy(abuf,
                                hs_hbm.at[pl.ds(base + half * _GK, _GK)])
                pltpu.sync_copy(bbuf,
                                hd_hbm.at[pl.ds(base + half * _GK, _GK)])
            return _

        lax.fori_loop(0, _G_PER_W // 1024, body, 0)

    return k


def _sc_gather(A, Bt, src2d, dst2d):
    return _sc_gather_kernel()(A, Bt, src2d, dst2d)


_SK = 1024
_S_PER_T = EP // 16
_S_CHUNKS = _S_PER_T // _SK
_ZROWS = NP // 16


@functools.lru_cache(maxsize=None)
def _sc_scatter_kernel():
    @functools.partial(
        pl.kernel,
        out_type=jax.ShapeDtypeStruct((NP, H), F32),
        mesh=_mesh(),
        compiler_params=pltpu.CompilerParams(use_tc_tiling_on_sc=False),
        scratch_types=[pltpu.VMEM((8, 128), jnp.int32),
                       pltpu.VMEM((_SK, 16), F32),
                       pltpu.VMEM_SHARED((NP, 16), F32)],
    )
    def k(e_hbm, dst_hbm, z_hbm, out_hbm, didx, ebuf, acc):
        c = lax.axis_index("c")
        s = lax.axis_index("s")
        rows = pl.ds(pl.multiple_of(s * _ZROWS, 64), _ZROWS)
        for half in range(2):
            cols = pl.ds(c * 32 + half * 16, 16)
            pltpu.sync_copy(z_hbm.at[rows, :], acc.at[rows, :])
            plsc.subcore_barrier()

            def body(j, _):
                base = pl.multiple_of(s * _S_PER_T + j * _SK, 1024)
                r = pl.multiple_of(base // 128, 8)
                pltpu.sync_copy(dst_hbm.at[pl.ds(r, 8), :], didx)
                pltpu.sync_copy(e_hbm.at[pl.ds(base, _SK), cols], ebuf)
                for t in range(8):
                    pltpu.sync_copy(ebuf.at[pl.ds(t * 128, 128), :],
                                    acc.at[didx.at[t]], add=True)
                return _

            lax.fori_loop(0, _S_CHUNKS, body, 0)
            plsc.subcore_barrier()
            pltpu.sync_copy(acc.at[rows, :], out_hbm.at[rows, cols])
            plsc.subcore_barrier()

    return k


def _sc_scatter(e, dst2d, z16):
    return _sc_scatter_kernel()(e, dst2d, z16)


_C_PER_T = EP // 32
_C_CHUNKS = _C_PER_T // _SK


@functools.lru_cache(maxsize=None)
def _sc_counts_kernel():
    @functools.partial(
        pl.kernel,
        out_type=jax.ShapeDtypeStruct((2, NP, 16), F32),
        mesh=_mesh(),
        compiler_params=pltpu.CompilerParams(use_tc_tiling_on_sc=False),
        scratch_types=[pltpu.VMEM((8, 128), jnp.int32),
                       pltpu.VMEM((_SK, 16), F32),
                       pltpu.VMEM_SHARED((NP, 16), F32)],
    )
    def k(dst_hbm, ones_hbm, z_hbm, out_hbm, didx, onesb, acc):
        c = lax.axis_index("c")
        s = lax.axis_index("s")
        rows = pl.ds(pl.multiple_of(s * _ZROWS, 64), _ZROWS)
        pltpu.sync_copy(ones_hbm, onesb)
        pltpu.sync_copy(z_hbm.at[rows, :], acc.at[rows, :])
        plsc.subcore_barrier()

        def body(j, _):
            base = pl.multiple_of(c * (EP // 2) + s * _C_PER_T + j * _SK,
                                  1024)
            r = pl.multiple_of(base // 128, 8)
            pltpu.sync_copy(dst_hbm.at[pl.ds(r, 8), :], didx)
            for t in range(8):
                pltpu.sync_copy(onesb.at[pl.ds(t * 128, 128), :],
                                acc.at[didx.at[t]], add=True)
            return _

        lax.fori_loop(0, _C_CHUNKS, body, 0)
        plsc.subcore_barrier()
        pltpu.sync_copy(acc.at[rows, :], out_hbm.at[c, rows, :])

    return k


def _sc_counts(dst2d, ones16, z16):
    return _sc_counts_kernel()(dst2d, ones16, z16)


def _b(v):
    return v.reshape(1, -1)


def _time_embed(t_idx):
    half = EMBED_DIM // 2
    freqs = jnp.exp(-np.log(10000.0)
                    * jnp.arange(half, dtype=F32) / half)
    ang = jnp.asarray(t_idx, F32).reshape(1, 1) * freqs[None, :]
    return jnp.concatenate([jnp.sin(ang), jnp.cos(ang)], axis=-1)


def kernel(X_t, t_idx, edge_index, edge_attr, node_graph_idx, n_graphs,
           node_features, rand_nodes, params):
    src = edge_index[0].astype(jnp.int32)
    dst = edge_index[1].astype(jnp.int32)
    src_p = jnp.concatenate([src, jnp.zeros((EP - E,), jnp.int32)])
    dst_p = jnp.concatenate([dst, jnp.full((EP - E,), DUMMY, jnp.int32)])
    src2d = src_p.reshape(EP // 128, 128)
    dst2d = dst_p.reshape(EP // 128, 128)
    ea_p = jnp.pad(edge_attr, ((0, EP - E), (0, 0)))

    t_emb = jnp.broadcast_to(_time_embed(t_idx), (N, EMBED_DIM))
    x = jnp.concatenate([X_t, node_features, t_emb, rand_nodes], axis=1)
    x = jnp.pad(x, ((0, NP - N), (0, H - x.shape[1])))

    g_p = jnp.pad(node_graph_idx.astype(jnp.int32), (0, NP - N),
                  constant_values=8)
    g3 = g_p.reshape(NBLK, 1, 1024)

    z16 = jnp.zeros((NP, 16), F32)
    ones16 = jnp.ones((_SK, 16), F32)
    delta = jnp.asarray(n_graphs - 8, F32).reshape(1)

    pr = params
    enc_w1 = jnp.pad(pr["enc_node"]["l1"]["W"], ((0, H - 53), (0, 0)))
    pass_w = []
    for p in pr["passes"]:
        we1 = p["edge"]["l1"]["W"]
        pass_w.append(dict(
            wa=we1[:H], wb=we1[H:2 * H], wc=we1[2 * H:],
            be1=_b(p["edge"]["l1"]["b"]),
            we2=p["edge"]["l2"]["W"], be2=_b(p["edge"]["l2"]["b"]),
            wn1h=p["node"]["l1"]["W"][:H], wn1a=p["node"]["l1"]["W"][H:],
            bn1=_b(p["node"]["l1"]["b"]),
            wn2=p["node"]["l2"]["W"], bn2=_b(p["node"]["l2"]["b"]),
        ))
    wml = jnp.concatenate([pr["mean_head"]["W"], pr["log_var_head"]["W"],
                           jnp.zeros((H, 4), F32)], axis=1)
    bml = jnp.concatenate([pr["mean_head"]["b"], pr["log_var_head"]["b"],
                           jnp.zeros((4,), F32)]).reshape(1, 8)
    vh = pr["value_head"]
    wv1 = jnp.pad(vh["l1"]["W"], ((0, 0), (0, 8)))
    bv1 = _b(jnp.pad(vh["l1"]["b"], (0, 8)))
    wv2 = jnp.pad(vh["l2"]["W"], ((0, 8), (0, 0)))
    bv2 = _b(vh["l2"]["b"])
    wv3 = jnp.pad(vh["l3"]["W"], ((0, 0), (0, 127)))
    bv3 = _b(jnp.pad(vh["l3"]["b"], (0, 127)))

    h, A, Bt = _tc_enc_node(
        x, enc_w1, _b(pr["enc_node"]["l1"]["b"]),
        pr["enc_node"]["l2"]["W"], _b(pr["enc_node"]["l2"]["b"]),
        pass_w[0]["wa"], pass_w[0]["be1"], pass_w[0]["wb"])
    e = _tc_enc_edge(
        ea_p, pr["enc_edge"]["l1"]["W"], _b(pr["enc_edge"]["l1"]["b"]),
        pr["enc_edge"]["l2"]["W"], _b(pr["enc_edge"]["l2"]["b"]))
    c2 = _sc_counts(dst2d, ones16, z16)

    for i in range(5):
        pw = pass_w[i]
        hs, hd = _sc_gather(A, Bt, src2d, dst2d)
        e = _tc_edge(hs, hd, e, pw["wc"], pw["we2"], pw["be2"])
        agg = _sc_scatter(e, dst2d, z16)
        if i < 4:
            nw = pass_w[i + 1]
            h, A, Bt = _tc_node(h, agg, c2, pw["wn1h"], pw["wn1a"],
                                pw["bn1"], pw["wn2"], pw["bn2"],
                                nw["wa"], nw["be1"], nw["wb"])
        else:
            emb = _tc_node_last(h, agg, c2, pw["wn1h"], pw["wn1a"],
                                pw["bn1"], pw["wn2"], pw["bn2"],
                                pr["dec"]["l1"]["W"], _b(pr["dec"]["l1"]["b"]),
                                pr["dec"]["l2"]["W"], _b(pr["dec"]["l2"]["b"]))

    out8, vals = _tc_head(emb, g3, wml, bml, wv1, bv1, wv2, bv2, wv3, bv3,
                          delta)
    pm = out8[:N, 0:2]
    plv = out8[:N, 2:4]
    values = vals[:, 0]
    return pm, plv, values, rand_nodes


# speedup vs baseline: 1.5377x; 1.0867x over previous
import functools

import jax
import jax.numpy as jnp
import numpy as np
from jax import lax
from jax.experimental import pallas as pl
from jax.experimental.pallas import tpu as pltpu
from jax.experimental.pallas import tpu_sc as plsc

N = 50000
NP = 50176
E = 800000
EP = 819200
H = 64
EMBED_DIM = 32
DUMMY = 50050
NBLK = NP // 1024
EBLK = EP // 1024

F32 = jnp.float32


def _dot(a, b):
    return jnp.dot(a, b, preferred_element_type=F32)


def _enc_node_body(x_ref, w1_ref, b1_ref, w2_ref, b2_ref, wa_ref, ba_ref,
                   wb_ref, h_ref, a_ref, bo_ref):
    x = x_ref[...]
    z = jax.nn.relu(_dot(x, w1_ref[...]) + b1_ref[...])
    h = _dot(z, w2_ref[...]) + b2_ref[...]
    h_ref[...] = h
    a_ref[...] = _dot(h, wa_ref[...]) + ba_ref[...]
    bo_ref[...] = _dot(h, wb_ref[...])


def _enc_edge_body(x_ref, w1_ref, b1_ref, w2_ref, b2_ref, out_ref):
    z = jax.nn.relu(_dot(x_ref[...], w1_ref[...]) + b1_ref[...])
    out_ref[...] = _dot(z, w2_ref[...]) + b2_ref[...]


def _edge_body(hs_ref, hd_ref, e_ref, w1c_ref, w2_ref, b2_ref, out_ref):
    e = e_ref[...]
    z = jax.nn.relu(hs_ref[...] + hd_ref[...] + _dot(e, w1c_ref[...]))
    out_ref[...] = e + _dot(z, w2_ref[...]) + b2_ref[...]


def _node_common(h_ref, agg_ref, c2_ref, wn1h_ref, wn1a_ref, bn1_ref,
                 wn2_ref, bn2_ref):
    h = h_ref[...]
    cnt = c2_ref[0, :, 0:1] + c2_ref[1, :, 0:1]
    denom = jnp.maximum(cnt, 1.0)
    aggm = agg_ref[...] / denom
    z = jax.nn.relu(_dot(h, wn1h_ref[...]) + _dot(aggm, wn1a_ref[...])
                    + bn1_ref[...])
    return h + _dot(z, wn2_ref[...]) + bn2_ref[...]


def _node_body(h_ref, agg_ref, c2_ref, wn1h_ref, wn1a_ref, bn1_ref, wn2_ref,
               bn2_ref, wa_ref, ba_ref, wb_ref, h_out, a_out, b_out):
    hn = _node_common(h_ref, agg_ref, c2_ref, wn1h_ref, wn1a_ref, bn1_ref,
                      wn2_ref, bn2_ref)
    h_out[...] = hn
    a_out[...] = _dot(hn, wa_ref[...]) + ba_ref[...]
    b_out[...] = _dot(hn, wb_ref[...])


def _node_last_body(h_ref, agg_ref, c2_ref, wn1h_ref, wn1a_ref, bn1_ref,
                    wn2_ref, bn2_ref, wd1_ref, bd1_ref, wd2_ref, bd2_ref,
                    emb_out):
    hn = _node_common(h_ref, agg_ref, c2_ref, wn1h_ref, wn1a_ref, bn1_ref,
                      wn2_ref, bn2_ref)
    z = jax.nn.relu(_dot(hn, wd1_ref[...]) + bd1_ref[...])
    emb_out[...] = _dot(z, wd2_ref[...]) + bd2_ref[...]


def _head_body(emb_ref, g_ref, wml_ref, bml_ref, wv1_ref, bv1_ref, wv2_ref,
               bv2_ref, wv3_ref, bv3_ref, delta_ref, out8_ref, vals_ref,
               ve_scr, np_scr):
    i = pl.program_id(0)
    emb = emb_ref[...]
    o = _dot(emb, wml_ref[...]) + bml_ref[...]
    lane = lax.broadcasted_iota(jnp.int32, o.shape, 1)
    is_lv = jnp.logical_and(lane >= 2, lane < 4)
    out8_ref[...] = jnp.where(is_lv, jnp.clip(o, -10.0, 2.0), o)

    g = g_ref[0]
    gid = lax.broadcasted_iota(jnp.int32, (8, g.shape[1]), 0)
    S = (gid == g).astype(F32)

    @pl.when(i == 0)
    def _():
        ve_scr[...] = jnp.zeros_like(ve_scr)
        np_scr[...] = jnp.zeros_like(np_scr)
        vals_ref[...] = jnp.zeros_like(vals_ref)

    ve_scr[...] += _dot(S, emb)
    np_scr[...] += jnp.broadcast_to(jnp.sum(S, axis=1, keepdims=True),
                                    np_scr.shape)

    @pl.when(i == NBLK - 1)
    def _():
        n_per = np_scr[...] + delta_ref[0]
        scale = jax.lax.rsqrt(jnp.maximum(n_per, 1.0))[:, :H]
        ven = ve_scr[...] * scale
        v1 = jax.nn.relu(_dot(ven, wv1_ref[...]) + bv1_ref[...])
        v2 = jax.nn.relu(_dot(v1, wv2_ref[...]) + bv2_ref[...])
        vals_ref[...] = _dot(v2, wv3_ref[...]) + bv3_ref[...]


def _row_spec(bm, bn):
    return pl.BlockSpec((bm, bn), lambda i: (i, 0))


def _w_spec(m, n):
    return pl.BlockSpec((m, n), lambda i: (0, 0))


def _tc_enc_node(x, w1, b1, w2, b2, wa, ba, wb):
    out = [jax.ShapeDtypeStruct((NP, H), F32)] * 3
    return pl.pallas_call(
        _enc_node_body,
        grid=(NBLK,),
        in_specs=[_row_spec(1024, H), _w_spec(H, H), _w_spec(1, H),
                  _w_spec(H, H), _w_spec(1, H), _w_spec(H, H), _w_spec(1, H),
                  _w_spec(H, H)],
        out_specs=[_row_spec(1024, H)] * 3,
        out_shape=out,
    )(x, w1, b1, w2, b2, wa, ba, wb)


def _tc_enc_edge(xp, w1, b1, w2, b2):
    return pl.pallas_call(
        _enc_edge_body,
        grid=(EBLK,),
        in_specs=[_row_spec(1024, 4), _w_spec(4, H), _w_spec(1, H),
                  _w_spec(H, H), _w_spec(1, H)],
        out_specs=_row_spec(1024, H),
        out_shape=jax.ShapeDtypeStruct((EP, H), F32),
    )(xp, w1, b1, w2, b2)


def _tc_edge(hs, hd, e, w1c, w2, b2):
    return pl.pallas_call(
        _edge_body,
        grid=(EBLK,),
        in_specs=[_row_spec(1024, H)] * 3 + [_w_spec(H, H), _w_spec(H, H),
                                             _w_spec(1, H)],
        out_specs=_row_spec(1024, H),
        out_shape=jax.ShapeDtypeStruct((EP, H), F32),
    )(hs, hd, e, w1c, w2, b2)


_C2_SPEC = pl.BlockSpec((2, 1024, 16), lambda i: (0, i, 0))


def _tc_node(h, agg, c2, wn1h, wn1a, bn1, wn2, bn2, wa, ba, wb):
    out = [jax.ShapeDtypeStruct((NP, H), F32)] * 3
    return pl.pallas_call(
        _node_body,
        grid=(NBLK,),
        in_specs=[_row_spec(1024, H), _row_spec(1024, H), _C2_SPEC,
                  _w_spec(H, H), _w_spec(H, H), _w_spec(1, H), _w_spec(H, H),
                  _w_spec(1, H), _w_spec(H, H), _w_spec(1, H), _w_spec(H, H)],
        out_specs=[_row_spec(1024, H)] * 3,
        out_shape=out,
    )(h, agg, c2, wn1h, wn1a, bn1, wn2, bn2, wa, ba, wb)


def _tc_node_last(h, agg, c2, wn1h, wn1a, bn1, wn2, bn2, wd1, bd1, wd2, bd2):
    return pl.pallas_call(
        _node_last_body,
        grid=(NBLK,),
        in_specs=[_row_spec(1024, H), _row_spec(1024, H), _C2_SPEC,
                  _w_spec(H, H), _w_spec(H, H), _w_spec(1, H), _w_spec(H, H),
                  _w_spec(1, H), _w_spec(H, H), _w_spec(1, H), _w_spec(H, H),
                  _w_spec(1, H)],
        out_specs=_row_spec(1024, H),
        out_shape=jax.ShapeDtypeStruct((NP, H), F32),
    )(h, agg, c2, wn1h, wn1a, bn1, wn2, bn2, wd1, bd1, wd2, bd2)


def _tc_head(emb, g3, wml, bml, wv1, bv1, wv2, bv2, wv3, bv3, delta):
    return pl.pallas_call(
        _head_body,
        grid=(NBLK,),
        in_specs=[_row_spec(1024, H),
                  pl.BlockSpec((1, 1, 1024), lambda i: (i, 0, 0)),
                  _w_spec(H, 8), _w_spec(1, 8), _w_spec(H, 128),
                  _w_spec(1, 128), _w_spec(128, H), _w_spec(1, H),
                  _w_spec(H, 128), _w_spec(1, 128),
                  pl.BlockSpec(memory_space=pltpu.SMEM)],
        out_specs=[_row_spec(1024, 8), _w_spec(8, 128)],
        out_shape=[jax.ShapeDtypeStruct((NP, 8), F32),
                   jax.ShapeDtypeStruct((8, 128), F32)],
        scratch_shapes=[pltpu.VMEM((8, H), F32), pltpu.VMEM((8, 128), F32)],
    )(emb, g3, wml, bml, wv1, bv1, wv2, bv2, wv3, bv3, delta)


def _mesh():
    return plsc.VectorSubcoreMesh(core_axis_name="c", subcore_axis_name="s",
                                  num_cores=2, num_subcores=16)


_GSUB = 256
_G_PER_W = EP // 32
_GM = 512
_G_MACROS = _G_PER_W // _GM


@functools.lru_cache(maxsize=None)
def _sc_gather_kernel():
    @functools.partial(
        pl.kernel,
        out_type=(jax.ShapeDtypeStruct((EP, H), F32),
                  jax.ShapeDtypeStruct((EP, H), F32)),
        mesh=_mesh(),
        compiler_params=pltpu.CompilerParams(use_tc_tiling_on_sc=False),
        scratch_types=[pltpu.VMEM((2, 4, 128), jnp.int32),
                       pltpu.VMEM((_GSUB, H), F32),
                       pltpu.VMEM((_GSUB, H), F32),
                       pltpu.VMEM((_GSUB, H), F32),
                       pltpu.VMEM((_GSUB, H), F32),
                       pltpu.SemaphoreType.DMA,
                       pltpu.SemaphoreType.DMA],
    )
    def k(a_hbm, b_hbm, ids_hbm, hs_hbm, hd_hbm,
          idxb, a0, a1, b0, b1, gsem, osem):
        wid = lax.axis_index("s") * 2 + lax.axis_index("c")
        wbase = wid * _G_PER_W
        abufs, bbufs = (a0, a1), (b0, b1)

        def macro(j, drain):
            base = pl.multiple_of(wbase + j * _GM, _GM)
            r = pl.multiple_of(base // 128, 4)
            pltpu.sync_copy(ids_hbm.at[:, pl.ds(r, 4), :], idxb)
            if drain:
                for buf in (a0, a1, b0, b1):
                    pltpu.make_async_copy(
                        a_hbm.at[pl.ds(0, _GSUB)], buf, osem).wait()
            cps = []
            for half in range(2):
                for t in range(2):
                    row = half * 2 + t
                    cps.append(pltpu.async_copy(
                        a_hbm.at[idxb.at[0, row]],
                        abufs[half].at[pl.ds(t * 128, 128)], gsem))
                    cps.append(pltpu.async_copy(
                        b_hbm.at[idxb.at[1, row]],
                        bbufs[half].at[pl.ds(t * 128, 128)], gsem))
            for cp in cps:
                cp.wait()
            for half in range(2):
                off = pl.ds(base + half * _GSUB, _GSUB)
                pltpu.async_copy(abufs[half], hs_hbm.at[off], osem)
                pltpu.async_copy(bbufs[half], hd_hbm.at[off], osem)

        macro(0, False)

        def body(j, carry):
            macro(j, True)
            return carry

        lax.fori_loop(1, _G_MACROS, body, 0)
        for buf in (a0, a1, b0, b1):
            pltpu.make_async_copy(a_hbm.at[pl.ds(0, _GSUB)], buf,
                                  osem).wait()

    return k


def _sc_gather(A, Bt, ids3):
    return _sc_gather_kernel()(A, Bt, ids3)


_SK = 1024
_S_PER_T = EP // 16
_S_CHUNKS = _S_PER_T // _SK
_ZROWS = NP // 16


@functools.lru_cache(maxsize=None)
def _sc_scatter_kernel():
    @functools.partial(
        pl.kernel,
        out_type=jax.ShapeDtypeStruct((NP, H), F32),
        mesh=_mesh(),
        compiler_params=pltpu.CompilerParams(use_tc_tiling_on_sc=False),
        scratch_types=[pltpu.VMEM((8, 128), jnp.int32),
                       pltpu.VMEM((8, 128), jnp.int32),
                       pltpu.VMEM((_SK, 16), F32),
                       pltpu.VMEM((_SK, 16), F32),
                       pltpu.VMEM_SHARED((NP, 16), F32),
                       pltpu.SemaphoreType.DMA,
                       pltpu.SemaphoreType.DMA],
    )
    def k(e_hbm, dst_hbm, z_hbm, out_hbm, d0, d1, e0, e1, acc, lsem, ssem):
        c = lax.axis_index("c")
        s = lax.axis_index("s")
        rows = pl.ds(pl.multiple_of(s * _ZROWS, 64), _ZROWS)
        dbufs, ebufs = (d0, d1), (e0, e1)
        tbase = s * _S_PER_T
        for half in range(2):
            cols = pl.ds(c * 32 + half * 16, 16)
            pltpu.sync_copy(z_hbm.at[rows, :], acc.at[rows, :])
            plsc.subcore_barrier()

            def load(j, bi):
                base = pl.multiple_of(tbase + j * _SK, 1024)
                r = pl.multiple_of(base // 128, 8)
                pltpu.async_copy(dst_hbm.at[pl.ds(r, 8), :], dbufs[bi],
                                 lsem)
                pltpu.async_copy(e_hbm.at[pl.ds(base, _SK), cols],
                                 ebufs[bi], lsem)

            def drain_loads(bi):
                pltpu.make_async_copy(dst_hbm.at[pl.ds(0, 8), :],
                                      dbufs[bi], lsem).wait()
                pltpu.make_async_copy(e_hbm.at[pl.ds(0, _SK), cols],
                                      ebufs[bi], lsem).wait()

            def scatter(bi):
                cps = [pltpu.async_copy(
                    ebufs[bi].at[pl.ds(t * 128, 128), :],
                    acc.at[dbufs[bi].at[t]], ssem, add=True)
                    for t in range(8)]
                for cp in cps:
                    cp.wait()

            load(0, 0)

            def body(js, carry):
                drain_loads(0)
                load(2 * js + 1, 1)
                scatter(0)
                drain_loads(1)

                @pl.when(js < _S_CHUNKS // 2 - 1)
                def _():
                    load(2 * js + 2, 0)

                scatter(1)
                return carry

            lax.fori_loop(0, _S_CHUNKS // 2, body, 0)
            plsc.subcore_barrier()
            pltpu.sync_copy(acc.at[rows, :], out_hbm.at[rows, cols])
            plsc.subcore_barrier()

    return k


def _sc_scatter(e, dst2d, z16):
    return _sc_scatter_kernel()(e, dst2d, z16)


_C_PER_T = EP // 32
_C_CHUNKS = _C_PER_T // _SK


@functools.lru_cache(maxsize=None)
def _sc_counts_kernel():
    @functools.partial(
        pl.kernel,
        out_type=jax.ShapeDtypeStruct((2, NP, 16), F32),
        mesh=_mesh(),
        compiler_params=pltpu.CompilerParams(use_tc_tiling_on_sc=False),
        scratch_types=[pltpu.VMEM((8, 128), jnp.int32),
                       pltpu.VMEM((_SK, 16), F32),
                       pltpu.VMEM_SHARED((NP, 16), F32)],
    )
    def k(dst_hbm, ones_hbm, z_hbm, out_hbm, didx, onesb, acc):
        c = lax.axis_index("c")
        s = lax.axis_index("s")
        rows = pl.ds(pl.multiple_of(s * _ZROWS, 64), _ZROWS)
        pltpu.sync_copy(ones_hbm, onesb)
        pltpu.sync_copy(z_hbm.at[rows, :], acc.at[rows, :])
        plsc.subcore_barrier()

        def body(j, _):
            base = pl.multiple_of(c * (EP // 2) + s * _C_PER_T + j * _SK,
                                  1024)
            r = pl.multiple_of(base // 128, 8)
            pltpu.sync_copy(dst_hbm.at[pl.ds(r, 8), :], didx)
            for t in range(8):
                pltpu.sync_copy(onesb.at[pl.ds(t * 128, 128), :],
                                acc.at[didx.at[t]], add=True)
            return _

        lax.fori_loop(0, _C_CHUNKS, body, 0)
        plsc.subcore_barrier()
        pltpu.sync_copy(acc.at[rows, :], out_hbm.at[c, rows, :])

    return k


def _sc_counts(dst2d, ones16, z16):
    return _sc_counts_kernel()(dst2d, ones16, z16)


def _b(v):
    return v.reshape(1, -1)


def _time_embed(t_idx):
    half = EMBED_DIM // 2
    freqs = jnp.exp(-np.log(10000.0)
                    * jnp.arange(half, dtype=F32) / half)
    ang = jnp.asarray(t_idx, F32).reshape(1, 1) * freqs[None, :]
    return jnp.concatenate([jnp.sin(ang), jnp.cos(ang)], axis=-1)


def kernel(X_t, t_idx, edge_index, edge_attr, node_graph_idx, n_graphs,
           node_features, rand_nodes, params):
    src = edge_index[0].astype(jnp.int32)
    dst = edge_index[1].astype(jnp.int32)
    src_p = jnp.concatenate([src, jnp.zeros((EP - E,), jnp.int32)])
    dst_p = jnp.concatenate([dst, jnp.full((EP - E,), DUMMY, jnp.int32)])
    ids3 = jnp.stack([src_p, dst_p]).reshape(2, EP // 128, 128)
    dst2d = dst_p.reshape(EP // 128, 128)
    ea_p = jnp.pad(edge_attr, ((0, EP - E), (0, 0)))

    t_emb = jnp.broadcast_to(_time_embed(t_idx), (N, EMBED_DIM))
    x = jnp.concatenate([X_t, node_features, t_emb, rand_nodes], axis=1)
    x = jnp.pad(x, ((0, NP - N), (0, H - x.shape[1])))

    g_p = jnp.pad(node_graph_idx.astype(jnp.int32), (0, NP - N),
                  constant_values=8)
    g3 = g_p.reshape(NBLK, 1, 1024)

    z16 = jnp.zeros((NP, 16), F32)
    ones16 = jnp.ones((_SK, 16), F32)
    delta = jnp.asarray(n_graphs - 8, F32).reshape(1)

    pr = params
    enc_w1 = jnp.pad(pr["enc_node"]["l1"]["W"], ((0, H - 53), (0, 0)))
    pass_w = []
    for p in pr["passes"]:
        we1 = p["edge"]["l1"]["W"]
        pass_w.append(dict(
            wa=we1[:H], wb=we1[H:2 * H], wc=we1[2 * H:],
            be1=_b(p["edge"]["l1"]["b"]),
            we2=p["edge"]["l2"]["W"], be2=_b(p["edge"]["l2"]["b"]),
            wn1h=p["node"]["l1"]["W"][:H], wn1a=p["node"]["l1"]["W"][H:],
            bn1=_b(p["node"]["l1"]["b"]),
            wn2=p["node"]["l2"]["W"], bn2=_b(p["node"]["l2"]["b"]),
        ))
    wml = jnp.concatenate([pr["mean_head"]["W"], pr["log_var_head"]["W"],
                           jnp.zeros((H, 4), F32)], axis=1)
    bml = jnp.concatenate([pr["mean_head"]["b"], pr["log_var_head"]["b"],
                           jnp.zeros((4,), F32)]).reshape(1, 8)
    vh = pr["value_head"]
    wv1 = jnp.pad(vh["l1"]["W"], ((0, 0), (0, 8)))
    bv1 = _b(jnp.pad(vh["l1"]["b"], (0, 8)))
    wv2 = jnp.pad(vh["l2"]["W"], ((0, 8), (0, 0)))
    bv2 = _b(vh["l2"]["b"])
    wv3 = jnp.pad(vh["l3"]["W"], ((0, 0), (0, 127)))
    bv3 = _b(jnp.pad(vh["l3"]["b"], (0, 127)))

    h, A, Bt = _tc_enc_node(
        x, enc_w1, _b(pr["enc_node"]["l1"]["b"]),
        pr["enc_node"]["l2"]["W"], _b(pr["enc_node"]["l2"]["b"]),
        pass_w[0]["wa"], pass_w[0]["be1"], pass_w[0]["wb"])
    e = _tc_enc_edge(
        ea_p, pr["enc_edge"]["l1"]["W"], _b(pr["enc_edge"]["l1"]["b"]),
        pr["enc_edge"]["l2"]["W"], _b(pr["enc_edge"]["l2"]["b"]))
    c2 = _sc_counts(dst2d, ones16, z16)

    for i in range(5):
        pw = pass_w[i]
        hs, hd = _sc_gather(A, Bt, ids3)
        e = _tc_edge(hs, hd, e, pw["wc"], pw["we2"], pw["be2"])
        agg = _sc_scatter(e, dst2d, z16)
        if i < 4:
            nw = pass_w[i + 1]
            h, A, Bt = _tc_node(h, agg, c2, pw["wn1h"], pw["wn1a"],
                                pw["bn1"], pw["wn2"], pw["bn2"],
                                nw["wa"], nw["be1"], nw["wb"])
        else:
            emb = _tc_node_last(h, agg, c2, pw["wn1h"], pw["wn1a"],
                                pw["bn1"], pw["wn2"], pw["bn2"],
                                pr["dec"]["l1"]["W"], _b(pr["dec"]["l1"]["b"]),
                                pr["dec"]["l2"]["W"], _b(pr["dec"]["l2"]["b"]))

    out8, vals = _tc_head(emb, g3, wml, bml, wv1, bv1, wv2, bv2, wv3, bv3,
                          delta)
    pm = out8[:N, 0:2]
    plv = out8[:N, 2:4]
    values = vals[:, 0]
    return pm, plv, values, rand_nodes


# speedup vs baseline: 1.6751x; 1.0894x over previous
import functools

import jax
import jax.numpy as jnp
import numpy as np
from jax import lax
from jax.experimental import pallas as pl
from jax.experimental.pallas import tpu as pltpu
from jax.experimental.pallas import tpu_sc as plsc

N = 50000
NP = 50176
E = 800000
EP = 819200
H = 64
EMBED_DIM = 32
DUMMY = 50050
NBLK = NP // 1024
EBLK = EP // 1024
EH = EP // 2
EHBLK = EH // 1024

F32 = jnp.float32


def _dot(a, b):
    return jnp.dot(a, b, preferred_element_type=F32)


def _enc_node_body(x_ref, w1_ref, b1_ref, w2_ref, b2_ref, wa_ref, ba_ref,
                   wb_ref, h_ref, a_ref, bo_ref):
    x = x_ref[...]
    z = jax.nn.relu(_dot(x, w1_ref[...]) + b1_ref[...])
    h = _dot(z, w2_ref[...]) + b2_ref[...]
    h_ref[...] = h
    a_ref[...] = _dot(h, wa_ref[...]) + ba_ref[...]
    bo_ref[...] = _dot(h, wb_ref[...])


def _enc_edge_body(x_ref, w1_ref, b1_ref, w2_ref, b2_ref, out_ref):
    z = jax.nn.relu(_dot(x_ref[...], w1_ref[...]) + b1_ref[...])
    out_ref[...] = _dot(z, w2_ref[...]) + b2_ref[...]


def _edge_body(hs_ref, hd_ref, e_ref, w1c_ref, w2_ref, b2_ref, out_ref):
    e = e_ref[...]
    z = jax.nn.relu(hs_ref[...] + hd_ref[...] + _dot(e, w1c_ref[...]))
    out_ref[...] = e + _dot(z, w2_ref[...]) + b2_ref[...]


def _node_common(h_ref, agg_ref, c2_ref, wn1h_ref, wn1a_ref, bn1_ref,
                 wn2_ref, bn2_ref):
    h = h_ref[...]
    cnt = c2_ref[0, :, 0:1] + c2_ref[1, :, 0:1]
    denom = jnp.maximum(cnt, 1.0)
    aggm = agg_ref[...] / denom
    z = jax.nn.relu(_dot(h, wn1h_ref[...]) + _dot(aggm, wn1a_ref[...])
                    + bn1_ref[...])
    return h + _dot(z, wn2_ref[...]) + bn2_ref[...]


def _node_body(h_ref, agg_ref, c2_ref, wn1h_ref, wn1a_ref, bn1_ref, wn2_ref,
               bn2_ref, wa_ref, ba_ref, wb_ref, h_out, a_out, b_out):
    hn = _node_common(h_ref, agg_ref, c2_ref, wn1h_ref, wn1a_ref, bn1_ref,
                      wn2_ref, bn2_ref)
    h_out[...] = hn
    a_out[...] = _dot(hn, wa_ref[...]) + ba_ref[...]
    b_out[...] = _dot(hn, wb_ref[...])


def _node_last_body(h_ref, agg_ref, c2_ref, wn1h_ref, wn1a_ref, bn1_ref,
                    wn2_ref, bn2_ref, wd1_ref, bd1_ref, wd2_ref, bd2_ref,
                    emb_out):
    hn = _node_common(h_ref, agg_ref, c2_ref, wn1h_ref, wn1a_ref, bn1_ref,
                      wn2_ref, bn2_ref)
    z = jax.nn.relu(_dot(hn, wd1_ref[...]) + bd1_ref[...])
    emb_out[...] = _dot(z, wd2_ref[...]) + bd2_ref[...]


def _head_body(emb_ref, g_ref, wml_ref, bml_ref, wv1_ref, bv1_ref, wv2_ref,
               bv2_ref, wv3_ref, bv3_ref, delta_ref, out8_ref, vals_ref,
               ve_scr, np_scr):
    i = pl.program_id(0)
    emb = emb_ref[...]
    o = _dot(emb, wml_ref[...]) + bml_ref[...]
    lane = lax.broadcasted_iota(jnp.int32, o.shape, 1)
    is_lv = jnp.logical_and(lane >= 2, lane < 4)
    out8_ref[...] = jnp.where(is_lv, jnp.clip(o, -10.0, 2.0), o)

    g = g_ref[0]
    gid = lax.broadcasted_iota(jnp.int32, (8, g.shape[1]), 0)
    S = (gid == g).astype(F32)

    @pl.when(i == 0)
    def _():
        ve_scr[...] = jnp.zeros_like(ve_scr)
        np_scr[...] = jnp.zeros_like(np_scr)
        vals_ref[...] = jnp.zeros_like(vals_ref)

    ve_scr[...] += _dot(S, emb)
    np_scr[...] += jnp.broadcast_to(jnp.sum(S, axis=1, keepdims=True),
                                    np_scr.shape)

    @pl.when(i == NBLK - 1)
    def _():
        n_per = np_scr[...] + delta_ref[0]
        scale = jax.lax.rsqrt(jnp.maximum(n_per, 1.0))[:, :H]
        ven = ve_scr[...] * scale
        v1 = jax.nn.relu(_dot(ven, wv1_ref[...]) + bv1_ref[...])
        v2 = jax.nn.relu(_dot(v1, wv2_ref[...]) + bv2_ref[...])
        vals_ref[...] = _dot(v2, wv3_ref[...]) + bv3_ref[...]


def _row_spec(bm, bn):
    return pl.BlockSpec((bm, bn), lambda i: (i, 0))


def _w_spec(m, n):
    return pl.BlockSpec((m, n), lambda i: (0, 0))


def _tc_enc_node(x, w1, b1, w2, b2, wa, ba, wb):
    out = [jax.ShapeDtypeStruct((NP, H), F32)] * 3
    return pl.pallas_call(
        _enc_node_body,
        grid=(NBLK,),
        in_specs=[_row_spec(1024, H), _w_spec(H, H), _w_spec(1, H),
                  _w_spec(H, H), _w_spec(1, H), _w_spec(H, H), _w_spec(1, H),
                  _w_spec(H, H)],
        out_specs=[_row_spec(1024, H)] * 3,
        out_shape=out,
    )(x, w1, b1, w2, b2, wa, ba, wb)


def _tc_enc_edge(xp, w1, b1, w2, b2):
    return pl.pallas_call(
        _enc_edge_body,
        grid=(EHBLK,),
        in_specs=[_row_spec(1024, 4), _w_spec(4, H), _w_spec(1, H),
                  _w_spec(H, H), _w_spec(1, H)],
        out_specs=_row_spec(1024, H),
        out_shape=jax.ShapeDtypeStruct((EH, H), F32),
    )(xp, w1, b1, w2, b2)


def _tc_edge(hs, hd, e, w1c, w2, b2):
    return pl.pallas_call(
        _edge_body,
        grid=(EHBLK,),
        in_specs=[_row_spec(1024, H)] * 3 + [_w_spec(H, H), _w_spec(H, H),
                                             _w_spec(1, H)],
        out_specs=_row_spec(1024, H),
        out_shape=jax.ShapeDtypeStruct((EH, H), F32),
    )(hs, hd, e, w1c, w2, b2)


_C2_SPEC = pl.BlockSpec((2, 1024, 16), lambda i: (0, i, 0))


def _tc_node(h, agg, c2, wn1h, wn1a, bn1, wn2, bn2, wa, ba, wb):
    out = [jax.ShapeDtypeStruct((NP, H), F32)] * 3
    return pl.pallas_call(
        _node_body,
        grid=(NBLK,),
        in_specs=[_row_spec(1024, H), _row_spec(1024, H), _C2_SPEC,
                  _w_spec(H, H), _w_spec(H, H), _w_spec(1, H), _w_spec(H, H),
                  _w_spec(1, H), _w_spec(H, H), _w_spec(1, H), _w_spec(H, H)],
        out_specs=[_row_spec(1024, H)] * 3,
        out_shape=out,
    )(h, agg, c2, wn1h, wn1a, bn1, wn2, bn2, wa, ba, wb)


def _tc_node_last(h, agg, c2, wn1h, wn1a, bn1, wn2, bn2, wd1, bd1, wd2, bd2):
    return pl.pallas_call(
        _node_last_body,
        grid=(NBLK,),
        in_specs=[_row_spec(1024, H), _row_spec(1024, H), _C2_SPEC,
                  _w_spec(H, H), _w_spec(H, H), _w_spec(1, H), _w_spec(H, H),
                  _w_spec(1, H), _w_spec(H, H), _w_spec(1, H), _w_spec(H, H),
                  _w_spec(1, H)],
        out_specs=_row_spec(1024, H),
        out_shape=jax.ShapeDtypeStruct((NP, H), F32),
    )(h, agg, c2, wn1h, wn1a, bn1, wn2, bn2, wd1, bd1, wd2, bd2)


def _tc_head(emb, g3, wml, bml, wv1, bv1, wv2, bv2, wv3, bv3, delta):
    return pl.pallas_call(
        _head_body,
        grid=(NBLK,),
        in_specs=[_row_spec(1024, H),
                  pl.BlockSpec((1, 1, 1024), lambda i: (i, 0, 0)),
                  _w_spec(H, 8), _w_spec(1, 8), _w_spec(H, 128),
                  _w_spec(1, 128), _w_spec(128, H), _w_spec(1, H),
                  _w_spec(H, 128), _w_spec(1, 128),
                  pl.BlockSpec(memory_space=pltpu.SMEM)],
        out_specs=[_row_spec(1024, 8), _w_spec(8, 128)],
        out_shape=[jax.ShapeDtypeStruct((NP, 8), F32),
                   jax.ShapeDtypeStruct((8, 128), F32)],
        scratch_shapes=[pltpu.VMEM((8, H), F32), pltpu.VMEM((8, 128), F32)],
    )(emb, g3, wml, bml, wv1, bv1, wv2, bv2, wv3, bv3, delta)


def _mesh():
    return plsc.VectorSubcoreMesh(core_axis_name="c", subcore_axis_name="s",
                                  num_cores=2, num_subcores=16)


_GSUB = 256
_G_PER_W = EH // 32
_GM = 512
_G_MACROS = _G_PER_W // _GM


@functools.lru_cache(maxsize=None)
def _sc_gather_kernel():
    @functools.partial(
        pl.kernel,
        out_type=(jax.ShapeDtypeStruct((EH, H), F32),
                  jax.ShapeDtypeStruct((EH, H), F32)),
        mesh=_mesh(),
        compiler_params=pltpu.CompilerParams(use_tc_tiling_on_sc=False),
        scratch_types=[pltpu.VMEM((2, 4, 128), jnp.int32),
                       pltpu.VMEM((_GSUB, H), F32),
                       pltpu.VMEM((_GSUB, H), F32),
                       pltpu.VMEM((_GSUB, H), F32),
                       pltpu.VMEM((_GSUB, H), F32),
                       pltpu.SemaphoreType.DMA,
                       pltpu.SemaphoreType.DMA],
    )
    def k(a_hbm, b_hbm, ids_hbm, hs_hbm, hd_hbm,
          idxb, a0, a1, b0, b1, gsem, osem):
        wid = lax.axis_index("s") * 2 + lax.axis_index("c")
        wbase = wid * _G_PER_W
        abufs, bbufs = (a0, a1), (b0, b1)

        def macro(j, drain):
            base = pl.multiple_of(wbase + j * _GM, _GM)
            r = pl.multiple_of(base // 128, 4)
            pltpu.sync_copy(ids_hbm.at[:, pl.ds(r, 4), :], idxb)
            if drain:
                for buf in (a0, a1, b0, b1):
                    pltpu.make_async_copy(
                        a_hbm.at[pl.ds(0, _GSUB)], buf, osem).wait()
            cps = []
            for half in range(2):
                for t in range(2):
                    row = half * 2 + t
                    cps.append(pltpu.async_copy(
                        a_hbm.at[idxb.at[0, row]],
                        abufs[half].at[pl.ds(t * 128, 128)], gsem))
                    cps.append(pltpu.async_copy(
                        b_hbm.at[idxb.at[1, row]],
                        bbufs[half].at[pl.ds(t * 128, 128)], gsem))
            for cp in cps:
                cp.wait()
            for half in range(2):
                off = pl.ds(base + half * _GSUB, _GSUB)
                pltpu.async_copy(abufs[half], hs_hbm.at[off], osem)
                pltpu.async_copy(bbufs[half], hd_hbm.at[off], osem)

        macro(0, False)

        def body(j, carry):
            macro(j, True)
            return carry

        lax.fori_loop(1, _G_MACROS, body, 0)
        for buf in (a0, a1, b0, b1):
            pltpu.make_async_copy(a_hbm.at[pl.ds(0, _GSUB)], buf,
                                  osem).wait()

    return k


def _sc_gather(A, Bt, ids3):
    return _sc_gather_kernel()(A, Bt, ids3)


_SK = 1024
_S_PER_T = EH // 16
_S_CHUNKS = _S_PER_T // _SK
_ZROWS = NP // 16


@functools.lru_cache(maxsize=None)
def _sc_scatter_kernel():
    @functools.partial(
        pl.kernel,
        out_type=jax.ShapeDtypeStruct((NP, H), F32),
        mesh=_mesh(),
        compiler_params=pltpu.CompilerParams(use_tc_tiling_on_sc=False),
        scratch_types=[pltpu.VMEM((8, 128), jnp.int32),
                       pltpu.VMEM((8, 128), jnp.int32),
                       pltpu.VMEM((_SK, 16), F32),
                       pltpu.VMEM((_SK, 16), F32),
                       pltpu.VMEM_SHARED((NP, 16), F32),
                       pltpu.SemaphoreType.DMA,
                       pltpu.SemaphoreType.DMA],
    )
    def k(e_hbm, dst_hbm, z_hbm, out_hbm, d0, d1, e0, e1, acc, lsem, ssem):
        c = lax.axis_index("c")
        s = lax.axis_index("s")
        rows = pl.ds(pl.multiple_of(s * _ZROWS, 64), _ZROWS)
        dbufs, ebufs = (d0, d1), (e0, e1)
        tbase = s * _S_PER_T
        nm = _S_CHUNKS
        for half in range(2):
            cols = pl.ds(c * 32 + half * 16, 16)
            pltpu.sync_copy(z_hbm.at[rows, cols], acc.at[rows, :])
            plsc.subcore_barrier()

            def load(j, bi):
                base = pl.multiple_of(tbase + j * _SK, 1024)
                r = pl.multiple_of(base // 128, 8)
                pltpu.async_copy(dst_hbm.at[pl.ds(r, 8), :], dbufs[bi],
                                 lsem)
                pltpu.async_copy(e_hbm.at[pl.ds(base, _SK), cols],
                                 ebufs[bi], lsem)

            def drain_loads(bi):
                pltpu.make_async_copy(dst_hbm.at[pl.ds(0, 8), :],
                                      dbufs[bi], lsem).wait()
                pltpu.make_async_copy(e_hbm.at[pl.ds(0, _SK), cols],
                                      ebufs[bi], lsem).wait()

            def scatter(bi):
                cps = [pltpu.async_copy(
                    ebufs[bi].at[pl.ds(t * 128, 128), :],
                    acc.at[dbufs[bi].at[t]], ssem, add=True)
                    for t in range(8)]
                for cp in cps:
                    cp.wait()

            load(0, 0)

            def body(js, carry):
                m1 = 2 * js + 1
                drain_loads(0)

                @pl.when(m1 < nm)
                def _():
                    load(m1, 1)

                scatter(0)

                @pl.when(m1 < nm)
                def _():
                    drain_loads(1)

                    @pl.when(m1 + 1 < nm)
                    def _():
                        load(m1 + 1, 0)

                    scatter(1)

                return carry

            lax.fori_loop(0, (nm + 1) // 2, body, 0)
            plsc.subcore_barrier()
            pltpu.sync_copy(acc.at[rows, :], out_hbm.at[rows, cols])
            plsc.subcore_barrier()

    return k


def _sc_scatter(e, dst2d, init64):
    return _sc_scatter_kernel()(e, dst2d, init64)


_C_PER_T = EP // 32
_C_CHUNKS = _C_PER_T // _SK


@functools.lru_cache(maxsize=None)
def _sc_counts_kernel():
    @functools.partial(
        pl.kernel,
        out_type=jax.ShapeDtypeStruct((2, NP, 16), F32),
        mesh=_mesh(),
        compiler_params=pltpu.CompilerParams(use_tc_tiling_on_sc=False),
        scratch_types=[pltpu.VMEM((8, 128), jnp.int32),
                       pltpu.VMEM((_SK, 16), F32),
                       pltpu.VMEM_SHARED((NP, 16), F32)],
    )
    def k(dst_hbm, ones_hbm, z_hbm, out_hbm, didx, onesb, acc):
        c = lax.axis_index("c")
        s = lax.axis_index("s")
        rows = pl.ds(pl.multiple_of(s * _ZROWS, 64), _ZROWS)
        pltpu.sync_copy(ones_hbm, onesb)
        pltpu.sync_copy(z_hbm.at[rows, :], acc.at[rows, :])
        plsc.subcore_barrier()

        def body(j, _):
            base = pl.multiple_of(c * (EP // 2) + s * _C_PER_T + j * _SK,
                                  1024)
            r = pl.multiple_of(base // 128, 8)
            pltpu.sync_copy(dst_hbm.at[pl.ds(r, 8), :], didx)
            for t in range(8):
                pltpu.sync_copy(onesb.at[pl.ds(t * 128, 128), :],
                                acc.at[didx.at[t]], add=True)
            return _

        lax.fori_loop(0, _C_CHUNKS, body, 0)
        plsc.subcore_barrier()
        pltpu.sync_copy(acc.at[rows, :], out_hbm.at[c, rows, :])

    return k


def _sc_counts(dst2d, ones16, z16):
    return _sc_counts_kernel()(dst2d, ones16, z16)


def _b(v):
    return v.reshape(1, -1)


def _time_embed(t_idx):
    half = EMBED_DIM // 2
    freqs = jnp.exp(-np.log(10000.0)
                    * jnp.arange(half, dtype=F32) / half)
    ang = jnp.asarray(t_idx, F32).reshape(1, 1) * freqs[None, :]
    return jnp.concatenate([jnp.sin(ang), jnp.cos(ang)], axis=-1)


def kernel(X_t, t_idx, edge_index, edge_attr, node_graph_idx, n_graphs,
           node_features, rand_nodes, params):
    src = edge_index[0].astype(jnp.int32)
    dst = edge_index[1].astype(jnp.int32)
    src_p = jnp.concatenate([src, jnp.zeros((EP - E,), jnp.int32)])
    dst_p = jnp.concatenate([dst, jnp.full((EP - E,), DUMMY, jnp.int32)])
    ids3h = [jnp.stack([src_p[h * EH:(h + 1) * EH],
                        dst_p[h * EH:(h + 1) * EH]]).reshape(2, EH // 128,
                                                             128)
             for h in range(2)]
    dst2dh = [dst_p[h * EH:(h + 1) * EH].reshape(EH // 128, 128)
              for h in range(2)]
    dst2d = dst_p.reshape(EP // 128, 128)
    ea_p = jnp.pad(edge_attr, ((0, EP - E), (0, 0)))

    t_emb = jnp.broadcast_to(_time_embed(t_idx), (N, EMBED_DIM))
    x = jnp.concatenate([X_t, node_features, t_emb, rand_nodes], axis=1)
    x = jnp.pad(x, ((0, NP - N), (0, H - x.shape[1])))

    g_p = jnp.pad(node_graph_idx.astype(jnp.int32), (0, NP - N),
                  constant_values=8)
    g3 = g_p.reshape(NBLK, 1, 1024)

    z16 = jnp.zeros((NP, 16), F32)
    z64 = jnp.zeros((NP, H), F32)
    ones16 = jnp.ones((_SK, 16), F32)
    delta = jnp.asarray(n_graphs - 8, F32).reshape(1)

    pr = params
    enc_w1 = jnp.pad(pr["enc_node"]["l1"]["W"], ((0, H - 53), (0, 0)))
    pass_w = []
    for p in pr["passes"]:
        we1 = p["edge"]["l1"]["W"]
        pass_w.append(dict(
            wa=we1[:H], wb=we1[H:2 * H], wc=we1[2 * H:],
            be1=_b(p["edge"]["l1"]["b"]),
            we2=p["edge"]["l2"]["W"], be2=_b(p["edge"]["l2"]["b"]),
            wn1h=p["node"]["l1"]["W"][:H], wn1a=p["node"]["l1"]["W"][H:],
            bn1=_b(p["node"]["l1"]["b"]),
            wn2=p["node"]["l2"]["W"], bn2=_b(p["node"]["l2"]["b"]),
        ))
    wml = jnp.concatenate([pr["mean_head"]["W"], pr["log_var_head"]["W"],
                           jnp.zeros((H, 4), F32)], axis=1)
    bml = jnp.concatenate([pr["mean_head"]["b"], pr["log_var_head"]["b"],
                           jnp.zeros((4,), F32)]).reshape(1, 8)
    vh = pr["value_head"]
    wv1 = jnp.pad(vh["l1"]["W"], ((0, 0), (0, 8)))
    bv1 = _b(jnp.pad(vh["l1"]["b"], (0, 8)))
    wv2 = jnp.pad(vh["l2"]["W"], ((0, 8), (0, 0)))
    bv2 = _b(vh["l2"]["b"])
    wv3 = jnp.pad(vh["l3"]["W"], ((0, 0), (0, 127)))
    bv3 = _b(jnp.pad(vh["l3"]["b"], (0, 127)))

    h, A, Bt = _tc_enc_node(
        x, enc_w1, _b(pr["enc_node"]["l1"]["b"]),
        pr["enc_node"]["l2"]["W"], _b(pr["enc_node"]["l2"]["b"]),
        pass_w[0]["wa"], pass_w[0]["be1"], pass_w[0]["wb"])
    eh = [_tc_enc_edge(
        ea_p[h * EH:(h + 1) * EH],
        pr["enc_edge"]["l1"]["W"], _b(pr["enc_edge"]["l1"]["b"]),
        pr["enc_edge"]["l2"]["W"], _b(pr["enc_edge"]["l2"]["b"]))
        for h in range(2)]
    c2 = _sc_counts(dst2d, ones16, z16)

    for i in range(5):
        pw = pass_w[i]
        hs0, hd0 = _sc_gather(A, Bt, ids3h[0])
        hs1, hd1 = _sc_gather(A, Bt, ids3h[1])
        eh[0] = _tc_edge(hs0, hd0, eh[0], pw["wc"], pw["we2"], pw["be2"])
        eh[1] = _tc_edge(hs1, hd1, eh[1], pw["wc"], pw["we2"], pw["be2"])
        agg0 = _sc_scatter(eh[0], dst2dh[0], z64)
        agg = _sc_scatter(eh[1], dst2dh[1], agg0)
        if i < 4:
            nw = pass_w[i + 1]
            h, A, Bt = _tc_node(h, agg, c2, pw["wn1h"], pw["wn1a"],
                                pw["bn1"], pw["wn2"], pw["bn2"],
                                nw["wa"], nw["be1"], nw["wb"])
        else:
            emb = _tc_node_last(h, agg, c2, pw["wn1h"], pw["wn1a"],
                                pw["bn1"], pw["wn2"], pw["bn2"],
                                pr["dec"]["l1"]["W"], _b(pr["dec"]["l1"]["b"]),
                                pr["dec"]["l2"]["W"], _b(pr["dec"]["l2"]["b"]))

    out8, vals = _tc_head(emb, g3, wml, bml, wv1, bv1, wv2, bv2, wv3, bv3,
                          delta)
    pm = out8[:N, 0:2]
    plv = out8[:N, 2:4]
    values = vals[:, 0]
    return pm, plv, values, rand_nodes


# speedup vs baseline: 1.6752x; 1.0001x over previous
import functools

import jax
import jax.numpy as jnp
import numpy as np
from jax import lax
from jax.experimental import pallas as pl
from jax.experimental.pallas import tpu as pltpu
from jax.experimental.pallas import tpu_sc as plsc

N = 50000
NP = 50176
E = 800000
EP = 819200
H = 64
EMBED_DIM = 32
DUMMY = 50050
NBLK = NP // 1024
EBLK = EP // 1024
EH = EP // 2
EHBLK = EH // 1024

F32 = jnp.float32


def _dot(a, b):
    return jnp.dot(a, b, preferred_element_type=F32)


def _enc_node_body(x_ref, w1_ref, b1_ref, w2_ref, b2_ref, wa_ref, ba_ref,
                   wb_ref, h_ref, a_ref, bo_ref):
    x = x_ref[...]
    z = jax.nn.relu(_dot(x, w1_ref[...]) + b1_ref[...])
    h = _dot(z, w2_ref[...]) + b2_ref[...]
    h_ref[...] = h
    a_ref[...] = _dot(h, wa_ref[...]) + ba_ref[...]
    bo_ref[...] = _dot(h, wb_ref[...])


def _enc_edge_body(x_ref, w1_ref, b1_ref, w2_ref, b2_ref, out_ref):
    z = jax.nn.relu(_dot(x_ref[...], w1_ref[...]) + b1_ref[...])
    out_ref[...] = _dot(z, w2_ref[...]) + b2_ref[...]


def _edge_body(hs_ref, hd_ref, e_ref, w1c_ref, w2_ref, b2_ref, out_ref):
    e = e_ref[...]
    z = jax.nn.relu(hs_ref[...] + hd_ref[...] + _dot(e, w1c_ref[...]))
    out_ref[...] = e + _dot(z, w2_ref[...]) + b2_ref[...]


def _node_common(h_ref, agg_ref, c2_ref, wn1h_ref, wn1a_ref, bn1_ref,
                 wn2_ref, bn2_ref):
    h = h_ref[...]
    cnt = c2_ref[0, :, 0:1] + c2_ref[1, :, 0:1]
    denom = jnp.maximum(cnt, 1.0)
    aggm = agg_ref[...] / denom
    z = jax.nn.relu(_dot(h, wn1h_ref[...]) + _dot(aggm, wn1a_ref[...])
                    + bn1_ref[...])
    return h + _dot(z, wn2_ref[...]) + bn2_ref[...]


def _node_body(h_ref, agg_ref, c2_ref, wn1h_ref, wn1a_ref, bn1_ref, wn2_ref,
               bn2_ref, wa_ref, ba_ref, wb_ref, h_out, a_out, b_out):
    hn = _node_common(h_ref, agg_ref, c2_ref, wn1h_ref, wn1a_ref, bn1_ref,
                      wn2_ref, bn2_ref)
    h_out[...] = hn
    a_out[...] = _dot(hn, wa_ref[...]) + ba_ref[...]
    b_out[...] = _dot(hn, wb_ref[...])


def _node_last_body(h_ref, agg_ref, c2_ref, wn1h_ref, wn1a_ref, bn1_ref,
                    wn2_ref, bn2_ref, wd1_ref, bd1_ref, wd2_ref, bd2_ref,
                    emb_out):
    hn = _node_common(h_ref, agg_ref, c2_ref, wn1h_ref, wn1a_ref, bn1_ref,
                      wn2_ref, bn2_ref)
    z = jax.nn.relu(_dot(hn, wd1_ref[...]) + bd1_ref[...])
    emb_out[...] = _dot(z, wd2_ref[...]) + bd2_ref[...]


def _head_body(emb_ref, g_ref, wml_ref, bml_ref, wv1_ref, bv1_ref, wv2_ref,
               bv2_ref, wv3_ref, bv3_ref, delta_ref, out8_ref, vals_ref,
               ve_scr, np_scr):
    i = pl.program_id(0)
    emb = emb_ref[...]
    o = _dot(emb, wml_ref[...]) + bml_ref[...]
    lane = lax.broadcasted_iota(jnp.int32, o.shape, 1)
    is_lv = jnp.logical_and(lane >= 2, lane < 4)
    out8_ref[...] = jnp.where(is_lv, jnp.clip(o, -10.0, 2.0), o)

    g = g_ref[0]
    gid = lax.broadcasted_iota(jnp.int32, (8, g.shape[1]), 0)
    S = (gid == g).astype(F32)

    @pl.when(i == 0)
    def _():
        ve_scr[...] = jnp.zeros_like(ve_scr)
        np_scr[...] = jnp.zeros_like(np_scr)
        vals_ref[...] = jnp.zeros_like(vals_ref)

    ve_scr[...] += _dot(S, emb)
    np_scr[...] += jnp.broadcast_to(jnp.sum(S, axis=1, keepdims=True),
                                    np_scr.shape)

    @pl.when(i == NBLK - 1)
    def _():
        n_per = np_scr[...] + delta_ref[0]
        scale = jax.lax.rsqrt(jnp.maximum(n_per, 1.0))[:, :H]
        ven = ve_scr[...] * scale
        v1 = jax.nn.relu(_dot(ven, wv1_ref[...]) + bv1_ref[...])
        v2 = jax.nn.relu(_dot(v1, wv2_ref[...]) + bv2_ref[...])
        vals_ref[...] = _dot(v2, wv3_ref[...]) + bv3_ref[...]


def _row_spec(bm, bn):
    return pl.BlockSpec((bm, bn), lambda i: (i, 0))


def _w_spec(m, n):
    return pl.BlockSpec((m, n), lambda i: (0, 0))


def _tc_enc_node(x, w1, b1, w2, b2, wa, ba, wb):
    out = [jax.ShapeDtypeStruct((NP, H), F32)] * 3
    return pl.pallas_call(
        _enc_node_body,
        grid=(NBLK,),
        in_specs=[_row_spec(1024, H), _w_spec(H, H), _w_spec(1, H),
                  _w_spec(H, H), _w_spec(1, H), _w_spec(H, H), _w_spec(1, H),
                  _w_spec(H, H)],
        out_specs=[_row_spec(1024, H)] * 3,
        out_shape=out,
    )(x, w1, b1, w2, b2, wa, ba, wb)


def _tc_enc_edge(xp, w1, b1, w2, b2):
    return pl.pallas_call(
        _enc_edge_body,
        grid=(EHBLK,),
        in_specs=[_row_spec(1024, 4), _w_spec(4, H), _w_spec(1, H),
                  _w_spec(H, H), _w_spec(1, H)],
        out_specs=_row_spec(1024, H),
        out_shape=jax.ShapeDtypeStruct((EH, H), F32),
    )(xp, w1, b1, w2, b2)


def _tc_edge(hs, hd, e, w1c, w2, b2):
    return pl.pallas_call(
        _edge_body,
        grid=(EHBLK,),
        in_specs=[_row_spec(1024, H)] * 3 + [_w_spec(H, H), _w_spec(H, H),
                                             _w_spec(1, H)],
        out_specs=_row_spec(1024, H),
        out_shape=jax.ShapeDtypeStruct((EH, H), F32),
    )(hs, hd, e, w1c, w2, b2)


_C2_SPEC = pl.BlockSpec((2, 1024, 16), lambda i: (0, i, 0))


def _tc_node(h, agg, c2, wn1h, wn1a, bn1, wn2, bn2, wa, ba, wb):
    out = [jax.ShapeDtypeStruct((NP, H), F32)] * 3
    return pl.pallas_call(
        _node_body,
        grid=(NBLK,),
        in_specs=[_row_spec(1024, H), _row_spec(1024, H), _C2_SPEC,
                  _w_spec(H, H), _w_spec(H, H), _w_spec(1, H), _w_spec(H, H),
                  _w_spec(1, H), _w_spec(H, H), _w_spec(1, H), _w_spec(H, H)],
        out_specs=[_row_spec(1024, H)] * 3,
        out_shape=out,
    )(h, agg, c2, wn1h, wn1a, bn1, wn2, bn2, wa, ba, wb)


def _tc_node_last(h, agg, c2, wn1h, wn1a, bn1, wn2, bn2, wd1, bd1, wd2, bd2):
    return pl.pallas_call(
        _node_last_body,
        grid=(NBLK,),
        in_specs=[_row_spec(1024, H), _row_spec(1024, H), _C2_SPEC,
                  _w_spec(H, H), _w_spec(H, H), _w_spec(1, H), _w_spec(H, H),
                  _w_spec(1, H), _w_spec(H, H), _w_spec(1, H), _w_spec(H, H),
                  _w_spec(1, H)],
        out_specs=_row_spec(1024, H),
        out_shape=jax.ShapeDtypeStruct((NP, H), F32),
    )(h, agg, c2, wn1h, wn1a, bn1, wn2, bn2, wd1, bd1, wd2, bd2)


def _tc_head(emb, g3, wml, bml, wv1, bv1, wv2, bv2, wv3, bv3, delta):
    return pl.pallas_call(
        _head_body,
        grid=(NBLK,),
        in_specs=[_row_spec(1024, H),
                  pl.BlockSpec((1, 1, 1024), lambda i: (i, 0, 0)),
                  _w_spec(H, 8), _w_spec(1, 8), _w_spec(H, 128),
                  _w_spec(1, 128), _w_spec(128, H), _w_spec(1, H),
                  _w_spec(H, 128), _w_spec(1, 128),
                  pl.BlockSpec(memory_space=pltpu.SMEM)],
        out_specs=[_row_spec(1024, 8), _w_spec(8, 128)],
        out_shape=[jax.ShapeDtypeStruct((NP, 8), F32),
                   jax.ShapeDtypeStruct((8, 128), F32)],
        scratch_shapes=[pltpu.VMEM((8, H), F32), pltpu.VMEM((8, 128), F32)],
    )(emb, g3, wml, bml, wv1, bv1, wv2, bv2, wv3, bv3, delta)


def _mesh():
    return plsc.VectorSubcoreMesh(core_axis_name="c", subcore_axis_name="s",
                                  num_cores=2, num_subcores=16)


_GSUB = 256
_G_PER_W = EH // 32
_GM = 512
_G_MACROS = _G_PER_W // _GM


@functools.lru_cache(maxsize=None)
def _sc_gather_kernel():
    @functools.partial(
        pl.kernel,
        out_type=(jax.ShapeDtypeStruct((EH, H), F32),
                  jax.ShapeDtypeStruct((EH, H), F32)),
        mesh=_mesh(),
        compiler_params=pltpu.CompilerParams(use_tc_tiling_on_sc=False),
        scratch_types=[pltpu.VMEM((2, 4, 128), jnp.int32),
                       pltpu.VMEM((_GSUB, H), F32),
                       pltpu.VMEM((_GSUB, H), F32),
                       pltpu.VMEM((_GSUB, H), F32),
                       pltpu.VMEM((_GSUB, H), F32),
                       pltpu.SemaphoreType.DMA,
                       pltpu.SemaphoreType.DMA],
    )
    def k(a_hbm, b_hbm, ids_hbm, hs_hbm, hd_hbm,
          idxb, a0, a1, b0, b1, gsem, osem):
        wid = lax.axis_index("s") * 2 + lax.axis_index("c")
        wbase = wid * _G_PER_W
        abufs, bbufs = (a0, a1), (b0, b1)

        def macro(j, drain):
            base = pl.multiple_of(wbase + j * _GM, _GM)
            r = pl.multiple_of(base // 128, 4)
            pltpu.sync_copy(ids_hbm.at[:, pl.ds(r, 4), :], idxb)
            if drain:
                for buf in (a0, a1, b0, b1):
                    pltpu.make_async_copy(
                        a_hbm.at[pl.ds(0, _GSUB)], buf, osem).wait()
            cps = []
            for half in range(2):
                for t in range(2):
                    row = half * 2 + t
                    cps.append(pltpu.async_copy(
                        a_hbm.at[idxb.at[0, row]],
                        abufs[half].at[pl.ds(t * 128, 128)], gsem))
                    cps.append(pltpu.async_copy(
                        b_hbm.at[idxb.at[1, row]],
                        bbufs[half].at[pl.ds(t * 128, 128)], gsem))
            for cp in cps:
                cp.wait()
            for half in range(2):
                off = pl.ds(base + half * _GSUB, _GSUB)
                pltpu.async_copy(abufs[half], hs_hbm.at[off], osem)
                pltpu.async_copy(bbufs[half], hd_hbm.at[off], osem)

        macro(0, False)

        def body(j, carry):
            macro(j, True)
            return carry

        lax.fori_loop(1, _G_MACROS, body, 0)
        for buf in (a0, a1, b0, b1):
            pltpu.make_async_copy(a_hbm.at[pl.ds(0, _GSUB)], buf,
                                  osem).wait()

    return k


def _sc_gather(A, Bt, ids3):
    return _sc_gather_kernel()(A, Bt, ids3)


_SK = 1024
_S_PER_T = EH // 16
_S_CHUNKS = _S_PER_T // _SK
_ZROWS = NP // 16


@functools.lru_cache(maxsize=None)
def _sc_scatter_kernel():
    @functools.partial(
        pl.kernel,
        out_type=jax.ShapeDtypeStruct((NP, H), F32),
        mesh=_mesh(),
        compiler_params=pltpu.CompilerParams(use_tc_tiling_on_sc=False),
        scratch_types=[pltpu.VMEM((8, 128), jnp.int32),
                       pltpu.VMEM((8, 128), jnp.int32),
                       pltpu.VMEM((_SK, 16), F32),
                       pltpu.VMEM((_SK, 16), F32),
                       pltpu.VMEM_SHARED((NP, 16), F32),
                       pltpu.SemaphoreType.DMA,
                       pltpu.SemaphoreType.DMA],
    )
    def k(e_hbm, dst_hbm, z_hbm, out_hbm, d0, d1, e0, e1, acc, lsem, ssem):
        c = lax.axis_index("c")
        s = lax.axis_index("s")
        rows = pl.ds(pl.multiple_of(s * _ZROWS, 64), _ZROWS)
        dbufs, ebufs = (d0, d1), (e0, e1)
        tbase = s * _S_PER_T
        nm = _S_CHUNKS
        for half in range(2):
            cols = pl.ds(c * 32 + half * 16, 16)
            pltpu.sync_copy(z_hbm.at[rows, cols], acc.at[rows, :])
            plsc.subcore_barrier()

            def load(j, bi):
                base = pl.multiple_of(tbase + j * _SK, 1024)
                r = pl.multiple_of(base // 128, 8)
                pltpu.async_copy(dst_hbm.at[pl.ds(r, 8), :], dbufs[bi],
                                 lsem)
                pltpu.async_copy(e_hbm.at[pl.ds(base, _SK), cols],
                                 ebufs[bi], lsem)

            def drain_loads(bi):
                pltpu.make_async_copy(dst_hbm.at[pl.ds(0, 8), :],
                                      dbufs[bi], lsem).wait()
                pltpu.make_async_copy(e_hbm.at[pl.ds(0, _SK), cols],
                                      ebufs[bi], lsem).wait()

            def scatter(bi):
                cps = [pltpu.async_copy(
                    ebufs[bi].at[pl.ds(t * 128, 128), :],
                    acc.at[dbufs[bi].at[t]], ssem, add=True)
                    for t in range(8)]
                for cp in cps:
                    cp.wait()

            load(0, 0)

            def body(js, carry):
                m1 = 2 * js + 1
                drain_loads(0)

                @pl.when(m1 < nm)
                def _():
                    load(m1, 1)

                scatter(0)

                @pl.when(m1 < nm)
                def _():
                    drain_loads(1)

                    @pl.when(m1 + 1 < nm)
                    def _():
                        load(m1 + 1, 0)

                    scatter(1)

                return carry

            lax.fori_loop(0, (nm + 1) // 2, body, 0)
            plsc.subcore_barrier()
            pltpu.sync_copy(acc.at[rows, :], out_hbm.at[rows, cols])
            plsc.subcore_barrier()

    return k


def _sc_scatter(e, dst2d, init64):
    return _sc_scatter_kernel()(e, dst2d, init64)


_C_PER_T = EP // 32
_C_CHUNKS = _C_PER_T // _SK


@functools.lru_cache(maxsize=None)
def _sc_counts_kernel():
    @functools.partial(
        pl.kernel,
        out_type=jax.ShapeDtypeStruct((2, NP, 16), F32),
        mesh=_mesh(),
        compiler_params=pltpu.CompilerParams(use_tc_tiling_on_sc=False),
        scratch_types=[pltpu.VMEM((8, 128), jnp.int32),
                       pltpu.VMEM((_SK, 16), F32),
                       pltpu.VMEM_SHARED((NP, 16), F32),
                       pltpu.SemaphoreType.DMA],
    )
    def k(dst_hbm, ones_hbm, z_hbm, out_hbm, didx, onesb, acc, csem):
        c = lax.axis_index("c")
        s = lax.axis_index("s")
        rows = pl.ds(pl.multiple_of(s * _ZROWS, 64), _ZROWS)
        pltpu.sync_copy(ones_hbm, onesb)
        pltpu.sync_copy(z_hbm.at[rows, :], acc.at[rows, :])
        plsc.subcore_barrier()

        def body(j, _):
            base = pl.multiple_of(c * (EP // 2) + s * _C_PER_T + j * _SK,
                                  1024)
            r = pl.multiple_of(base // 128, 8)
            pltpu.sync_copy(dst_hbm.at[pl.ds(r, 8), :], didx)
            cps = [pltpu.async_copy(onesb.at[pl.ds(t * 128, 128), :],
                                    acc.at[didx.at[t]], csem, add=True)
                   for t in range(8)]
            for cp in cps:
                cp.wait()
            return _

        lax.fori_loop(0, _C_CHUNKS, body, 0)
        plsc.subcore_barrier()
        pltpu.sync_copy(acc.at[rows, :], out_hbm.at[c, rows, :])

    return k


def _sc_counts(dst2d, ones16, z16):
    return _sc_counts_kernel()(dst2d, ones16, z16)


def _b(v):
    return v.reshape(1, -1)


def _time_embed(t_idx):
    half = EMBED_DIM // 2
    freqs = jnp.exp(-np.log(10000.0)
                    * jnp.arange(half, dtype=F32) / half)
    ang = jnp.asarray(t_idx, F32).reshape(1, 1) * freqs[None, :]
    return jnp.concatenate([jnp.sin(ang), jnp.cos(ang)], axis=-1)


def kernel(X_t, t_idx, edge_index, edge_attr, node_graph_idx, n_graphs,
           node_features, rand_nodes, params):
    src = edge_index[0].astype(jnp.int32)
    dst = edge_index[1].astype(jnp.int32)
    src_p = jnp.concatenate([src, jnp.zeros((EP - E,), jnp.int32)])
    dst_p = jnp.concatenate([dst, jnp.full((EP - E,), DUMMY, jnp.int32)])
    ids3h = [jnp.stack([src_p[h * EH:(h + 1) * EH],
                        dst_p[h * EH:(h + 1) * EH]]).reshape(2, EH // 128,
                                                             128)
             for h in range(2)]
    dst2dh = [dst_p[h * EH:(h + 1) * EH].reshape(EH // 128, 128)
              for h in range(2)]
    dst2d = dst_p.reshape(EP // 128, 128)
    ea_p = jnp.pad(edge_attr, ((0, EP - E), (0, 0)))

    t_emb = jnp.broadcast_to(_time_embed(t_idx), (N, EMBED_DIM))
    x = jnp.concatenate([X_t, node_features, t_emb, rand_nodes], axis=1)
    x = jnp.pad(x, ((0, NP - N), (0, H - x.shape[1])))

    g_p = jnp.pad(node_graph_idx.astype(jnp.int32), (0, NP - N),
                  constant_values=8)
    g3 = g_p.reshape(NBLK, 1, 1024)

    z16 = jnp.zeros((NP, 16), F32)
    z64 = jnp.zeros((NP, H), F32)
    ones16 = jnp.ones((_SK, 16), F32)
    delta = jnp.asarray(n_graphs - 8, F32).reshape(1)

    pr = params
    enc_w1 = jnp.pad(pr["enc_node"]["l1"]["W"], ((0, H - 53), (0, 0)))
    pass_w = []
    for p in pr["passes"]:
        we1 = p["edge"]["l1"]["W"]
        pass_w.append(dict(
            wa=we1[:H], wb=we1[H:2 * H], wc=we1[2 * H:],
            be1=_b(p["edge"]["l1"]["b"]),
            we2=p["edge"]["l2"]["W"], be2=_b(p["edge"]["l2"]["b"]),
            wn1h=p["node"]["l1"]["W"][:H], wn1a=p["node"]["l1"]["W"][H:],
            bn1=_b(p["node"]["l1"]["b"]),
            wn2=p["node"]["l2"]["W"], bn2=_b(p["node"]["l2"]["b"]),
        ))
    wml = jnp.concatenate([pr["mean_head"]["W"], pr["log_var_head"]["W"],
                           jnp.zeros((H, 4), F32)], axis=1)
    bml = jnp.concatenate([pr["mean_head"]["b"], pr["log_var_head"]["b"],
                           jnp.zeros((4,), F32)]).reshape(1, 8)
    vh = pr["value_head"]
    wv1 = jnp.pad(vh["l1"]["W"], ((0, 0), (0, 8)))
    bv1 = _b(jnp.pad(vh["l1"]["b"], (0, 8)))
    wv2 = jnp.pad(vh["l2"]["W"], ((0, 8), (0, 0)))
    bv2 = _b(vh["l2"]["b"])
    wv3 = jnp.pad(vh["l3"]["W"], ((0, 0), (0, 127)))
    bv3 = _b(jnp.pad(vh["l3"]["b"], (0, 127)))

    h, A, Bt = _tc_enc_node(
        x, enc_w1, _b(pr["enc_node"]["l1"]["b"]),
        pr["enc_node"]["l2"]["W"], _b(pr["enc_node"]["l2"]["b"]),
        pass_w[0]["wa"], pass_w[0]["be1"], pass_w[0]["wb"])
    eh = [_tc_enc_edge(
        ea_p[h * EH:(h + 1) * EH],
        pr["enc_edge"]["l1"]["W"], _b(pr["enc_edge"]["l1"]["b"]),
        pr["enc_edge"]["l2"]["W"], _b(pr["enc_edge"]["l2"]["b"]))
        for h in range(2)]
    c2 = _sc_counts(dst2d, ones16, z16)

    for i in range(5):
        pw = pass_w[i]
        hs0, hd0 = _sc_gather(A, Bt, ids3h[0])
        hs1, hd1 = _sc_gather(A, Bt, ids3h[1])
        eh[0] = _tc_edge(hs0, hd0, eh[0], pw["wc"], pw["we2"], pw["be2"])
        eh[1] = _tc_edge(hs1, hd1, eh[1], pw["wc"], pw["we2"], pw["be2"])
        agg0 = _sc_scatter(eh[0], dst2dh[0], z64)
        agg = _sc_scatter(eh[1], dst2dh[1], agg0)
        if i < 4:
            nw = pass_w[i + 1]
            h, A, Bt = _tc_node(h, agg, c2, pw["wn1h"], pw["wn1a"],
                                pw["bn1"], pw["wn2"], pw["bn2"],
                                nw["wa"], nw["be1"], nw["wb"])
        else:
            emb = _tc_node_last(h, agg, c2, pw["wn1h"], pw["wn1a"],
                                pw["bn1"], pw["wn2"], pw["bn2"],
                                pr["dec"]["l1"]["W"], _b(pr["dec"]["l1"]["b"]),
                                pr["dec"]["l2"]["W"], _b(pr["dec"]["l2"]["b"]))

    out8, vals = _tc_head(emb, g3, wml, bml, wv1, bv1, wv2, bv2, wv3, bv3,
                          delta)
    pm = out8[:N, 0:2]
    plv = out8[:N, 2:4]
    values = vals[:, 0]
    return pm, plv, values, rand_nodes
